# baseline (speedup 1.0000x reference)
import functools

import jax
import jax.numpy as jnp
from jax import lax
from jax.experimental import pallas as pl
from jax.experimental.pallas import tpu as pltpu
from jax.experimental.pallas import tpu_sc as plsc

B_ = 8
N_ = 10000
NP = 10240
E_ = 320000
EP = 327680
H_ = 128
BN = 2048
NC = 2
NS = 16
RPT = NP // NS
CH = 128
NCH = EP // NS // CH
NCHD = EP // (NC * NS) // CH
_BN_SCALE = 1.0 / (1.0 + 1e-5) ** 0.5


def _sc_mesh():
    return plsc.VectorSubcoreMesh(
        core_axis_name="c", subcore_axis_name="s", num_cores=NC, num_subcores=NS
    )


def _deg_body(col_hbm, consts_hbm, out_hbm, cbuf, ones, zbuf, acc):
    cid = lax.axis_index("c")
    tid = lax.axis_index("s")
    pltpu.sync_copy(consts_hbm.at[pl.ds(0, CH)], ones)
    pltpu.sync_copy(consts_hbm.at[pl.ds(CH, 64)], zbuf)
    for t in range(RPT // 64):
        pltpu.sync_copy(zbuf, acc.at[pl.ds(tid * RPT + t * 64, 64)])
    plsc.subcore_barrier()
    ebase = cid * (EP // NC) + tid * (EP // (NC * NS))

    def body(i, carry):
        pltpu.sync_copy(col_hbm.at[pl.ds(ebase + i * CH, CH)], cbuf)
        pltpu.sync_copy(ones, acc.at[cbuf], add=True)
        return carry

    lax.fori_loop(0, NCHD, body, 0)
    plsc.subcore_barrier()
    pltpu.sync_copy(
        acc.at[pl.ds(tid * RPT, RPT)],
        out_hbm.at[pl.ds(cid * NP + tid * RPT, RPT)],
    )


IBLK = 32
HF = H_ // 2


def _spmm_body(hlo_hbm, hhi_hbm, rowabs_hbm, col3_hbm, zeros_hbm,
               olo_hbm, ohi_hbm,
               rbuf, cbuf, g0, g1, zbuf, table, acc, gs0, gs1, ss0, ss1):
    cid = lax.axis_index("c")
    tid = lax.axis_index("s")
    gbufs = (g0, g1)
    gsems = (gs0, gs1)
    ssems = (ss0, ss1)
    pltpu.sync_copy(zeros_hbm, zbuf)

    def fire_gather(j, k, src):
        pltpu.async_copy(src.at[rbuf.at[j]], gbufs[k], gsems[k])

    def wait_gather(j, k, src):
        pltpu.make_async_copy(src.at[rbuf.at[j]], gbufs[k],
                              gsems[k]).wait()

    def fire_scatter(j, k):
        pltpu.async_copy(gbufs[k], acc.at[cbuf.at[j]], ssems[k], add=True)

    def wait_scatter(j, k):
        pltpu.make_async_copy(gbufs[k], acc.at[cbuf.at[j]], ssems[k]).wait()

    for p in range(B_ // NC):
        b = cid + NC * p
        for hf in range(2):
            src_hbm = (hlo_hbm, hhi_hbm)[hf]
            dst_hbm = (olo_hbm, ohi_hbm)[hf]
            pltpu.sync_copy(
                src_hbm.at[pl.ds(b * NP + tid * RPT, RPT)],
                table.at[pl.ds(tid * RPT, RPT)],
            )
            for t in range(RPT // IBLK):
                pltpu.sync_copy(zbuf,
                                acc.at[pl.ds(tid * RPT + t * IBLK, IBLK)])
            plsc.subcore_barrier()

            def blk_body(blk, carry, src_hbm=src_hbm):
                pltpu.sync_copy(rowabs_hbm.at[b, tid, pl.ds(blk * IBLK, IBLK)],
                                rbuf)
                pltpu.sync_copy(col3_hbm.at[tid, pl.ds(blk * IBLK, IBLK)],
                                cbuf)
                fire_gather(0, 0, src_hbm)
                fire_gather(1, 1, src_hbm)

                def body(i2, carry2):
                    for k in range(2):
                        i = i2 * 2 + k
                        wait_gather(i, k, src_hbm)
                        fire_scatter(i, k)

                        @pl.when(i2 < IBLK // 2 - 1)
                        def _():
                            wait_scatter(i, k)
                            fire_gather(i + 2, k, src_hbm)
                    return carry2

                lax.fori_loop(0, IBLK // 2, body, 0)
                wait_scatter(IBLK - 2, 0)
                wait_scatter(IBLK - 1, 1)
                return carry

            lax.fori_loop(0, NCH // IBLK, blk_body, 0)
            plsc.subcore_barrier()
            pltpu.sync_copy(
                acc.at[pl.ds(tid * RPT, RPT)],
                dst_hbm.at[pl.ds(b * NP + tid * RPT, RPT)],
            )
            plsc.subcore_barrier()


_SC_BUILT = {}


def _deg_sc(col):
    if "deg" not in _SC_BUILT:
        _SC_BUILT["deg"] = functools.partial(
            pl.kernel,
            out_type=jax.ShapeDtypeStruct((NC * NP, 16), jnp.float32),
            mesh=_sc_mesh(),
            scratch_types=[
                pltpu.VMEM((CH,), jnp.int32),
                pltpu.VMEM((CH, 16), jnp.float32),
                pltpu.VMEM((64, 16), jnp.float32),
                pltpu.VMEM_SHARED((NP, 16), jnp.float32),
            ],
        )(_deg_body)
    consts = jnp.concatenate(
        [jnp.ones((CH, 16), jnp.float32), jnp.zeros((64, 16), jnp.float32)], 0)
    return _SC_BUILT["deg"](col, consts)


def _spmm_sc(hlo, hhi, row3, col3):
    if "spmm" not in _SC_BUILT:
        _SC_BUILT["spmm"] = functools.partial(
            pl.kernel,
            out_type=[jax.ShapeDtypeStruct((B_ * NP, HF), jnp.float32),
                      jax.ShapeDtypeStruct((B_ * NP, HF), jnp.float32)],
            mesh=_sc_mesh(),
            compiler_params=pltpu.CompilerParams(use_tc_tiling_on_sc=False),
            scratch_types=[
                pltpu.VMEM((IBLK, CH), jnp.int32),
                pltpu.VMEM((IBLK, CH), jnp.int32),
                pltpu.VMEM((CH, HF), jnp.float32),
                pltpu.VMEM((CH, HF), jnp.float32),
                pltpu.VMEM((IBLK, HF), jnp.float32),
                pltpu.VMEM_SHARED((NP, HF), jnp.float32),
                pltpu.VMEM_SHARED((NP, HF), jnp.float32),
                pltpu.SemaphoreType.DMA,
                pltpu.SemaphoreType.DMA,
                pltpu.SemaphoreType.DMA,
                pltpu.SemaphoreType.DMA,
            ],
        )(_spmm_body)
    zeros = jnp.zeros((IBLK, HF), jnp.float32)
    return _SC_BUILT["spmm"](hlo, hhi, row3, col3, zeros)


def _prep_body(x_ref, da_ref, W1_ref, b1_ref, W2_ref, b2_ref, Wc0_ref,
               h_ref, hlo_ref, hhi_ref, dinv_ref):
    xv = x_ref[0, 0, :]
    t = jnp.maximum(xv[:, None] * W1_ref[0][None, :] + b1_ref[0][None, :], 0.0)
    h = jnp.dot(t, W2_ref[...], preferred_element_type=jnp.float32)
    h = h + b2_ref[0][None, :]
    deg = da_ref[0, :, 0] + da_ref[1, :, 0] + 1.0
    dinv = lax.rsqrt(deg)
    hw = jnp.dot(h, Wc0_ref[...], preferred_element_type=jnp.float32)
    h_ref[0] = h
    hws = dinv[:, None] * hw
    hlo_ref[0] = hws[:, :HF]
    hhi_ref[0] = hws[:, HF:]
    dinv_ref[...] = dinv


def _mid_body(alo_ref, ahi_ref, hlo_ref, hhi_ref, h_ref, dinv_ref, g_ref,
              bt_ref, bc_ref, Wc1_ref, h1_ref, h1lo_ref, h1hi_ref):
    dinv = dinv_ref[...]
    s = jnp.concatenate([alo_ref[0] + hlo_ref[0], ahi_ref[0] + hhi_ref[0]],
                        axis=-1)
    t = dinv[:, None] * s + bc_ref[0][None, :]
    t = t * (g_ref[0][None, :] * _BN_SCALE) + bt_ref[0][None, :]
    h1 = jnp.maximum(t, 0.0) + h_ref[0]
    hw1 = jnp.dot(h1, Wc1_ref[...], preferred_element_type=jnp.float32)
    h1_ref[0] = h1
    hws1 = dinv[:, None] * hw1
    h1lo_ref[0] = hws1[:, :HF]
    h1hi_ref[0] = hws1[:, HF:]


def _pool_body(alo_ref, ahi_ref, hlo_ref, hhi_ref, h1_ref, dinv_ref, g_ref,
               bt_ref, bc_ref, out_ref):
    n = pl.program_id(1)
    dinv = dinv_ref[...]
    s = jnp.concatenate([alo_ref[0] + hlo_ref[0], ahi_ref[0] + hhi_ref[0]],
                        axis=-1)
    t = dinv[:, None] * s + bc_ref[0][None, :]
    t = t * (g_ref[0][None, :] * _BN_SCALE) + bt_ref[0][None, :]
    h2 = jnp.maximum(t, 0.0) + h1_ref[0]
    iot = lax.broadcasted_iota(jnp.int32, (BN, 1), 0)
    h2 = jnp.where(iot < (N_ - n * BN), h2, 0.0)
    part = jnp.sum(h2, axis=0)

    @pl.when(n == 0)
    def _():
        out_ref[0, 0, :] = part

    @pl.when(n > 0)
    def _():
        out_ref[0, 0, :] = out_ref[0, 0, :] + part


def _proj_body(p_ref, Wp_ref, bp_ref, z_ref):
    z = jnp.dot(p_ref[...] * (1.0 / N_), Wp_ref[...],
                preferred_element_type=jnp.float32)
    z_ref[...] = z + bp_ref[0][None, :]


def _full(shape):
    return pl.BlockSpec(shape, lambda b, n: tuple(0 for _ in shape))


def kernel(x, edge_index, W1, b1, W2, b2, Wc0, bc0, Wc1, bc1, g0, bt0,
                 g1, bt1, Wp, bp):
    f32 = jnp.float32
    xp = jnp.pad(x, ((0, 0), (0, NP - N_))).reshape(B_, 1, NP)
    row = jnp.pad(edge_index[0], (0, EP - E_), constant_values=NP - 1)
    col = jnp.pad(edge_index[1], (0, EP - E_), constant_values=NP - 1)
    row3 = row.reshape(NS, NCH, CH)
    rowabs = (row.reshape(1, NS, NCH, CH)
              + (jnp.arange(B_, dtype=jnp.int32) * NP)[:, None, None, None])
    col3 = col.reshape(NS, NCH, CH)
    b1r, b2r = b1.reshape(1, -1), b2.reshape(1, -1)
    bc0r, bc1r = bc0.reshape(1, -1), bc1.reshape(1, -1)
    g0r, g1r = g0.reshape(1, -1), g1.reshape(1, -1)
    bt0r, bt1r = bt0.reshape(1, -1), bt1.reshape(1, -1)
    bpr = bp.reshape(1, -1)

    degacc = _deg_sc(col).reshape(NC, NP, 16)

    grid = (B_, NP // BN)
    node3 = pl.BlockSpec((1, BN, H_), lambda b, n: (b, n, 0))
    half3 = pl.BlockSpec((1, BN, HF), lambda b, n: (b, n, 0))
    dinv_spec = pl.BlockSpec((BN,), lambda b, n: (n,))
    half_sds = jax.ShapeDtypeStruct((B_, NP, HF), f32)
    h, hws0lo, hws0hi, dinv = pl.pallas_call(
        _prep_body,
        grid=grid,
        in_specs=[
            pl.BlockSpec((1, 1, BN), lambda b, n: (b, 0, n)),
            pl.BlockSpec((NC, BN, 16), lambda b, n: (0, n, 0)),
            _full((1, 64)), _full((1, 64)), _full((64, H_)), _full((1, H_)),
            _full((H_, H_)),
        ],
        out_specs=[node3, half3, half3, dinv_spec],
        out_shape=[
            jax.ShapeDtypeStruct((B_, NP, H_), f32),
            half_sds, half_sds,
            jax.ShapeDtypeStruct((NP,), f32),
        ],
    )(xp, degacc, W1, b1r, W2, b2r, Wc0)

    a0lo, a0hi = _spmm_sc(hws0lo.reshape(B_ * NP, HF),
                          hws0hi.reshape(B_ * NP, HF), rowabs, col3)
    a0lo = a0lo.reshape(B_, NP, HF)
    a0hi = a0hi.reshape(B_, NP, HF)

    h1, hws1lo, hws1hi = pl.pallas_call(
        _mid_body,
        grid=grid,
        in_specs=[
            half3, half3, half3, half3, node3, dinv_spec,
            _full((1, H_)), _full((1, H_)), _full((1, H_)), _full((H_, H_)),
        ],
        out_specs=[node3, half3, half3],
        out_shape=[
            jax.ShapeDtypeStruct((B_, NP, H_), f32),
            half_sds, half_sds,
        ],
    )(a0lo, a0hi, hws0lo, hws0hi, h, dinv, g0r, bt0r, bc0r, Wc1)

    a1lo, a1hi = _spmm_sc(hws1lo.reshape(B_ * NP, HF),
                          hws1hi.reshape(B_ * NP, HF), rowabs, col3)
    a1lo = a1lo.reshape(B_, NP, HF)
    a1hi = a1hi.reshape(B_, NP, HF)

    pooled = pl.pallas_call(
        _pool_body,
        grid=grid,
        in_specs=[
            half3, half3, half3, half3, node3, dinv_spec,
            _full((1, H_)), _full((1, H_)), _full((1, H_)),
        ],
        out_specs=pl.BlockSpec((1, 1, H_), lambda b, n: (b, 0, 0)),
        out_shape=jax.ShapeDtypeStruct((B_, 1, H_), f32),
    )(a1lo, a1hi, hws1lo, hws1hi, h1, dinv, g1r, bt1r, bc1r)

    z = pl.pallas_call(
        _proj_body,
        grid=(1, 1),
        in_specs=[_full((B_, H_)), _full((H_, H_)), _full((1, H_))],
        out_specs=_full((B_, H_)),
        out_shape=jax.ShapeDtypeStruct((B_, H_), f32),
    )(pooled.reshape(B_, H_), Wp, bpr)
    return z


# speedup vs baseline: 1.4922x; 1.4922x over previous
import functools

import jax
import jax.numpy as jnp
from jax import lax
from jax.experimental import pallas as pl
from jax.experimental.pallas import tpu as pltpu
from jax.experimental.pallas import tpu_sc as plsc

B_ = 8
N_ = 10000
NP = 10240
E_ = 320000
EP = 327680
H_ = 128
BN = 2048
NC = 2
NS = 16
RPT = NP // NS
CH = 128
NCH = EP // NS // CH
NCHD = EP // (NC * NS) // CH
_BN_SCALE = 1.0 / (1.0 + 1e-5) ** 0.5


def _sc_mesh():
    return plsc.VectorSubcoreMesh(
        core_axis_name="c", subcore_axis_name="s", num_cores=NC, num_subcores=NS
    )


def _deg_body(col_hbm, consts_hbm, out_hbm, cbuf, ones, zbuf, acc):
    cid = lax.axis_index("c")
    tid = lax.axis_index("s")
    pltpu.sync_copy(consts_hbm.at[pl.ds(0, CH)], ones)
    pltpu.sync_copy(consts_hbm.at[pl.ds(CH, 64)], zbuf)
    for t in range(RPT // 64):
        pltpu.sync_copy(zbuf, acc.at[pl.ds(tid * RPT + t * 64, 64)])
    plsc.subcore_barrier()
    ebase = cid * (EP // NC) + tid * (EP // (NC * NS))

    def body(i, carry):
        pltpu.sync_copy(col_hbm.at[pl.ds(ebase + i * CH, CH)], cbuf)
        pltpu.sync_copy(ones, acc.at[cbuf], add=True)
        return carry

    lax.fori_loop(0, NCHD, body, 0)
    plsc.subcore_barrier()
    pltpu.sync_copy(
        acc.at[pl.ds(tid * RPT, RPT)],
        out_hbm.at[pl.ds(cid * NP + tid * RPT, RPT)],
    )


HF = H_ // 2
WIN = 40
TPW = 26
HPW = 14
ZR = 40


def _spmm_body(hws_hbm, rowmix_hbm, col3_hbm, zeros_hbm, out_hbm,
               rbuf, cbuf, gT0, gT1, gH0, gH1, zbuf, table, acc,
               sgT0, sgT1, sgH0, sgH1, ssT0, ssT1, ssH0, ssH1):
    cid = lax.axis_index("c")
    tid = lax.axis_index("s")
    gb = {"T": (gT0, gT1), "H": (gH0, gH1)}
    gs = {"T": (sgT0, sgT1), "H": (sgH0, sgH1)}
    ss = {"T": (ssT0, ssT1), "H": (ssH0, ssH1)}
    pltpu.sync_copy(zeros_hbm, zbuf)

    steps = []
    for j in range(HPW - 1):
        steps += [("T", 2 * j), ("T", 2 * j + 1), ("H", j)]
    steps += [("T", 2 * (HPW - 1) + k) for k in range(TPW - 2 * (HPW - 1))]
    steps += [("H", HPW - 1)]
    nsrc = {"T": TPW, "H": HPW}
    srcs = {"T": table, "H": hws_hbm}

    def loc(src, c):
        return c if src == "T" else TPW + c

    def fire_g(src, c):
        k = c % 2
        pltpu.async_copy(srcs[src].at[rbuf.at[loc(src, c)]],
                         gb[src][k], gs[src][k])

    def wait_g(src, c):
        k = c % 2
        pltpu.make_async_copy(srcs[src].at[rbuf.at[loc(src, c)]],
                              gb[src][k], gs[src][k]).wait()

    def fire_s(src, c):
        k = c % 2
        pltpu.async_copy(gb[src][k], acc.at[cbuf.at[loc(src, c)]],
                         ss[src][k], add=True)

    def wait_s(src, c):
        k = c % 2
        pltpu.make_async_copy(gb[src][k], acc.at[cbuf.at[loc(src, c)]],
                              ss[src][k]).wait()

    def pass_body(q, carry):
        hf = q % 2
        b = cid + NC * (q // 2)
        base = hf * (B_ * NP) + b * NP
        pltpu.sync_copy(
            hws_hbm.at[pl.ds(base + tid * RPT, RPT)],
            table.at[pl.ds(tid * RPT, RPT)],
        )
        for t in range(RPT // ZR):
            pltpu.sync_copy(zbuf, acc.at[pl.ds(tid * RPT + t * ZR, ZR)])
        plsc.subcore_barrier()

        def sb_body(sb, carry2):
            pltpu.sync_copy(
                rowmix_hbm.at[hf, b, tid, pl.ds(sb * WIN, WIN)], rbuf)
            pltpu.sync_copy(
                col3_hbm.at[tid, pl.ds(sb * WIN, WIN)], cbuf)
            for src, c in (("T", 0), ("T", 1), ("H", 0), ("H", 1)):
                fire_g(src, c)
            for src, c in steps:
                wait_g(src, c)
                fire_s(src, c)
                if c + 2 < nsrc[src]:
                    wait_s(src, c)
                    fire_g(src, c + 2)
            for src in ("T", "H"):
                wait_s(src, nsrc[src] - 2)
                wait_s(src, nsrc[src] - 1)
            return carry2

        lax.fori_loop(0, NCH // WIN, sb_body, 0)
        plsc.subcore_barrier()
        pltpu.sync_copy(
            acc.at[pl.ds(tid * RPT, RPT)],
            out_hbm.at[pl.ds(base + tid * RPT, RPT)],
        )
        plsc.subcore_barrier()
        return carry

    lax.fori_loop(0, B_, pass_body, 0)


_SC_BUILT = {}


def _deg_sc(col):
    if "deg" not in _SC_BUILT:
        _SC_BUILT["deg"] = functools.partial(
            pl.kernel,
            out_type=jax.ShapeDtypeStruct((NC * NP, 16), jnp.float32),
            mesh=_sc_mesh(),
            scratch_types=[
                pltpu.VMEM((CH,), jnp.int32),
                pltpu.VMEM((CH, 16), jnp.float32),
                pltpu.VMEM((64, 16), jnp.float32),
                pltpu.VMEM_SHARED((NP, 16), jnp.float32),
            ],
        )(_deg_body)
    consts = jnp.concatenate(
        [jnp.ones((CH, 16), jnp.float32), jnp.zeros((64, 16), jnp.float32)], 0)
    return _SC_BUILT["deg"](col, consts)


def _spmm_sc(hwscat, rowmix, col3):
    if "spmm" not in _SC_BUILT:
        _SC_BUILT["spmm"] = functools.partial(
            pl.kernel,
            out_type=jax.ShapeDtypeStruct((2 * B_ * NP, HF), jnp.float32),
            mesh=_sc_mesh(),
            compiler_params=pltpu.CompilerParams(use_tc_tiling_on_sc=False),
            scratch_types=[
                pltpu.VMEM((WIN, CH), jnp.int32),
                pltpu.VMEM((WIN, CH), jnp.int32),
                pltpu.VMEM((CH, HF), jnp.float32),
                pltpu.VMEM((CH, HF), jnp.float32),
                pltpu.VMEM((CH, HF), jnp.float32),
                pltpu.VMEM((CH, HF), jnp.float32),
                pltpu.VMEM((ZR, HF), jnp.float32),
                pltpu.VMEM_SHARED((NP, HF), jnp.float32),
                pltpu.VMEM_SHARED((NP, HF), jnp.float32),
            ] + [pltpu.SemaphoreType.DMA] * 8,
        )(_spmm_body)
    zeros = jnp.zeros((ZR, HF), jnp.float32)
    return _SC_BUILT["spmm"](hwscat, rowmix, col3, zeros)


def _prep_body(x_ref, da_ref, W1_ref, b1_ref, W2_ref, b2_ref, Wc0_ref,
               h_ref, hws4_ref, dinv_ref):
    xv = x_ref[0, 0, :]
    t = jnp.maximum(xv[:, None] * W1_ref[0][None, :] + b1_ref[0][None, :], 0.0)
    h = jnp.dot(t, W2_ref[...], preferred_element_type=jnp.float32)
    h = h + b2_ref[0][None, :]
    deg = da_ref[0, :, 0] + da_ref[1, :, 0] + 1.0
    dinv = lax.rsqrt(deg)
    hw = jnp.dot(h, Wc0_ref[...], preferred_element_type=jnp.float32)
    h_ref[0] = h
    hws = dinv[:, None] * hw
    hws4_ref[0, 0] = hws[:, :HF]
    hws4_ref[1, 0] = hws[:, HF:]
    dinv_ref[...] = dinv


def _mid_body(a4_ref, hw4_ref, h_ref, dinv_ref, g_ref,
              bt_ref, bc_ref, Wc1_ref, h1_ref, hws14_ref):
    dinv = dinv_ref[...]
    s = jnp.concatenate([a4_ref[0, 0] + hw4_ref[0, 0],
                         a4_ref[1, 0] + hw4_ref[1, 0]], axis=-1)
    t = dinv[:, None] * s + bc_ref[0][None, :]
    t = t * (g_ref[0][None, :] * _BN_SCALE) + bt_ref[0][None, :]
    h1 = jnp.maximum(t, 0.0) + h_ref[0]
    hw1 = jnp.dot(h1, Wc1_ref[...], preferred_element_type=jnp.float32)
    h1_ref[0] = h1
    hws1 = dinv[:, None] * hw1
    hws14_ref[0, 0] = hws1[:, :HF]
    hws14_ref[1, 0] = hws1[:, HF:]


def _pool_body(a4_ref, hw4_ref, h1_ref, dinv_ref, g_ref,
               bt_ref, bc_ref, out_ref):
    n = pl.program_id(1)
    dinv = dinv_ref[...]
    s = jnp.concatenate([a4_ref[0, 0] + hw4_ref[0, 0],
                         a4_ref[1, 0] + hw4_ref[1, 0]], axis=-1)
    t = dinv[:, None] * s + bc_ref[0][None, :]
    t = t * (g_ref[0][None, :] * _BN_SCALE) + bt_ref[0][None, :]
    h2 = jnp.maximum(t, 0.0) + h1_ref[0]
    iot = lax.broadcasted_iota(jnp.int32, (BN, 1), 0)
    h2 = jnp.where(iot < (N_ - n * BN), h2, 0.0)
    part = jnp.sum(h2, axis=0)

    @pl.when(n == 0)
    def _():
        out_ref[0, 0, :] = part

    @pl.when(n > 0)
    def _():
        out_ref[0, 0, :] = out_ref[0, 0, :] + part


def _proj_body(p_ref, Wp_ref, bp_ref, z_ref):
    z = jnp.dot(p_ref[...] * (1.0 / N_), Wp_ref[...],
                preferred_element_type=jnp.float32)
    z_ref[...] = z + bp_ref[0][None, :]


def _full(shape):
    return pl.BlockSpec(shape, lambda b, n: tuple(0 for _ in shape))


def kernel(x, edge_index, W1, b1, W2, b2, Wc0, bc0, Wc1, bc1, g0, bt0,
                 g1, bt1, Wp, bp):
    f32 = jnp.float32
    xp = jnp.pad(x, ((0, 0), (0, NP - N_))).reshape(B_, 1, NP)
    row = jnp.pad(edge_index[0], (0, EP - E_), constant_values=NP - 1)
    col = jnp.pad(edge_index[1], (0, EP - E_), constant_values=NP - 1)
    row3 = row.reshape(NS, NCH, CH)
    col3 = col.reshape(NS, NCH, CH)
    tmask = (jnp.arange(NCH, dtype=jnp.int32) % WIN) < TPW
    offs = ((jnp.arange(2, dtype=jnp.int32) * B_ * NP)[:, None]
            + (jnp.arange(B_, dtype=jnp.int32) * NP)[None, :])
    rowmix = jnp.where(tmask[None, None, None, :, None],
                       row3[None, None],
                       row3[None, None] + offs[:, :, None, None, None])
    b1r, b2r = b1.reshape(1, -1), b2.reshape(1, -1)
    bc0r, bc1r = bc0.reshape(1, -1), bc1.reshape(1, -1)
    g0r, g1r = g0.reshape(1, -1), g1.reshape(1, -1)
    bt0r, bt1r = bt0.reshape(1, -1), bt1.reshape(1, -1)
    bpr = bp.reshape(1, -1)

    degacc = _deg_sc(col).reshape(NC, NP, 16)

    grid = (B_, NP // BN)
    node3 = pl.BlockSpec((1, BN, H_), lambda b, n: (b, n, 0))
    quad4 = pl.BlockSpec((2, 1, BN, HF), lambda b, n: (0, b, n, 0))
    dinv_spec = pl.BlockSpec((BN,), lambda b, n: (n,))
    quad_sds = jax.ShapeDtypeStruct((2, B_, NP, HF), f32)
    h, hws0, dinv = pl.pallas_call(
        _prep_body,
        grid=grid,
        in_specs=[
            pl.BlockSpec((1, 1, BN), lambda b, n: (b, 0, n)),
            pl.BlockSpec((NC, BN, 16), lambda b, n: (0, n, 0)),
            _full((1, 64)), _full((1, 64)), _full((64, H_)), _full((1, H_)),
            _full((H_, H_)),
        ],
        out_specs=[node3, quad4, dinv_spec],
        out_shape=[
            jax.ShapeDtypeStruct((B_, NP, H_), f32),
            quad_sds,
            jax.ShapeDtypeStruct((NP,), f32),
        ],
    )(xp, degacc, W1, b1r, W2, b2r, Wc0)

    a0 = _spmm_sc(hws0.reshape(2 * B_ * NP, HF), rowmix,
                  col3).reshape(2, B_, NP, HF)

    h1, hws1 = pl.pallas_call(
        _mid_body,
        grid=grid,
        in_specs=[
            quad4, quad4, node3, dinv_spec,
            _full((1, H_)), _full((1, H_)), _full((1, H_)), _full((H_, H_)),
        ],
        out_specs=[node3, quad4],
        out_shape=[
            jax.ShapeDtypeStruct((B_, NP, H_), f32),
            quad_sds,
        ],
    )(a0, hws0, h, dinv, g0r, bt0r, bc0r, Wc1)

    a1 = _spmm_sc(hws1.reshape(2 * B_ * NP, HF), rowmix,
                  col3).reshape(2, B_, NP, HF)

    pooled = pl.pallas_call(
        _pool_body,
        grid=grid,
        in_specs=[
            quad4, quad4, node3, dinv_spec,
            _full((1, H_)), _full((1, H_)), _full((1, H_)),
        ],
        out_specs=pl.BlockSpec((1, 1, H_), lambda b, n: (b, 0, 0)),
        out_shape=jax.ShapeDtypeStruct((B_, 1, H_), f32),
    )(a1, hws1, h1, dinv, g1r, bt1r, bc1r)

    z = pl.pallas_call(
        _proj_body,
        grid=(1, 1),
        in_specs=[_full((B_, H_)), _full((H_, H_)), _full((1, H_))],
        out_specs=_full((B_, H_)),
        out_shape=jax.ShapeDtypeStruct((B_, H_), f32),
    )(pooled.reshape(B_, H_), Wp, bpr)
    return z


# speedup vs baseline: 1.7609x; 1.1800x over previous
import functools

import jax
import jax.numpy as jnp
from jax import lax
from jax.experimental import pallas as pl
from jax.experimental.pallas import tpu as pltpu
from jax.experimental.pallas import tpu_sc as plsc

B_ = 8
N_ = 10000
NP = 10240
E_ = 320000
EP = 327680
H_ = 128
BN = 2048
NC = 2
NS = 16
RPT = NP // NS
CH = 128
NCH = EP // NS // CH
NCHD = EP // (NC * NS) // CH
_BN_SCALE = 1.0 / (1.0 + 1e-5) ** 0.5


def _sc_mesh():
    return plsc.VectorSubcoreMesh(
        core_axis_name="c", subcore_axis_name="s", num_cores=NC, num_subcores=NS
    )


def _deg_body(col_hbm, consts_hbm, out_hbm, cbuf, ones, zbuf, acc):
    cid = lax.axis_index("c")
    tid = lax.axis_index("s")
    pltpu.sync_copy(consts_hbm.at[pl.ds(0, CH)], ones)
    pltpu.sync_copy(consts_hbm.at[pl.ds(CH, 64)], zbuf)
    for t in range(RPT // 64):
        pltpu.sync_copy(zbuf, acc.at[pl.ds(tid * RPT + t * 64, 64)])
    plsc.subcore_barrier()
    ebase = cid * (EP // NC) + tid * (EP // (NC * NS))

    def body(i, carry):
        pltpu.sync_copy(col_hbm.at[pl.ds(ebase + i * CH, CH)], cbuf)
        pltpu.sync_copy(ones, acc.at[cbuf], add=True)
        return carry

    lax.fori_loop(0, NCHD, body, 0)
    plsc.subcore_barrier()
    pltpu.sync_copy(
        acc.at[pl.ds(tid * RPT, RPT)],
        out_hbm.at[pl.ds(cid * NP + tid * RPT, RPT)],
    )


IBLK = 32
HF = H_ // 2
ZR = 40


def _spmm_body(hlo_hbm, hhi_hbm, row3_hbm, col3_hbm, zeros_hbm,
               olo_hbm, ohi_hbm,
               rbuf, cbuf, g0, g1, g2, g3, zbuf, table, acc,
               gs0, gs1, gs2, gs3, ss0, ss1, ss2, ss3, stsem):
    cid = lax.axis_index("c")
    tid = lax.axis_index("s")
    gbufs = (g0, g1, g2, g3)
    gsems = (gs0, gs1, gs2, gs3)
    ssems = (ss0, ss1, ss2, ss3)
    pltpu.sync_copy(zeros_hbm, zbuf)

    def fire_gather(j, k, src):
        pltpu.async_copy(src.at[rbuf.at[j]], gbufs[k], gsems[k])

    def wait_gather(j, k, src):
        pltpu.make_async_copy(src.at[rbuf.at[j]], gbufs[k],
                              gsems[k]).wait()

    def fire_scatter(j, k):
        pltpu.async_copy(gbufs[k], acc.at[cbuf.at[j]], ssems[k], add=True)

    def wait_scatter(j, k):
        pltpu.make_async_copy(gbufs[k], acc.at[cbuf.at[j]], ssems[k]).wait()

    for p in range(B_ // NC):
        b = cid + NC * p
        for hf in range(2):
            src_hbm = (hlo_hbm, hhi_hbm)[hf]
            dst_hbm = (olo_hbm, ohi_hbm)[hf]
            pltpu.async_copy(
                src_hbm.at[pl.ds(b * NP + tid * RPT, RPT)],
                table.at[pl.ds(tid * RPT, RPT)],
                stsem,
            )
            for t in range(RPT // ZR):
                pltpu.sync_copy(zbuf,
                                acc.at[pl.ds(tid * RPT + t * ZR, ZR)])
            pltpu.make_async_copy(
                src_hbm.at[pl.ds(b * NP + tid * RPT, RPT)],
                table.at[pl.ds(tid * RPT, RPT)],
                stsem,
            ).wait()
            plsc.subcore_barrier()

            def blk_body(blk, carry):
                pltpu.sync_copy(row3_hbm.at[tid, pl.ds(blk * IBLK, IBLK)],
                                rbuf)
                pltpu.sync_copy(col3_hbm.at[tid, pl.ds(blk * IBLK, IBLK)],
                                cbuf)
                for k in range(4):
                    fire_gather(k, k, table)

                def body(i4, carry2):
                    for k in range(4):
                        i = i4 * 4 + k
                        wait_gather(i, k, table)
                        fire_scatter(i, k)

                        @pl.when(i4 < IBLK // 4 - 1)
                        def _():
                            wait_scatter(i, k)
                            fire_gather(i + 4, k, table)
                    return carry2

                lax.fori_loop(0, IBLK // 4, body, 0)
                for k in range(4):
                    wait_scatter(IBLK - 4 + k, k)
                return carry

            lax.fori_loop(0, NCH // IBLK, blk_body, 0)
            plsc.subcore_barrier()
            pltpu.sync_copy(
                acc.at[pl.ds(tid * RPT, RPT)],
                dst_hbm.at[pl.ds(b * NP + tid * RPT, RPT)],
            )


_SC_BUILT = {}


def _deg_sc(col):
    if "deg" not in _SC_BUILT:
        _SC_BUILT["deg"] = functools.partial(
            pl.kernel,
            out_type=jax.ShapeDtypeStruct((NC * NP, 16), jnp.float32),
            mesh=_sc_mesh(),
            scratch_types=[
                pltpu.VMEM((CH,), jnp.int32),
                pltpu.VMEM((CH, 16), jnp.float32),
                pltpu.VMEM((64, 16), jnp.float32),
                pltpu.VMEM_SHARED((NP, 16), jnp.float32),
            ],
        )(_deg_body)
    consts = jnp.concatenate(
        [jnp.ones((CH, 16), jnp.float32), jnp.zeros((64, 16), jnp.float32)], 0)
    return _SC_BUILT["deg"](col, consts)


def _spmm_sc(hlo, hhi, row3, col3):
    if "spmm" not in _SC_BUILT:
        _SC_BUILT["spmm"] = functools.partial(
            pl.kernel,
            out_type=[jax.ShapeDtypeStruct((B_ * NP, HF), jnp.float32),
                      jax.ShapeDtypeStruct((B_ * NP, HF), jnp.float32)],
            mesh=_sc_mesh(),
            compiler_params=pltpu.CompilerParams(use_tc_tiling_on_sc=False),
            scratch_types=[
                pltpu.VMEM((IBLK, CH), jnp.int32),
                pltpu.VMEM((IBLK, CH), jnp.int32),
                pltpu.VMEM((CH, HF), jnp.float32),
                pltpu.VMEM((CH, HF), jnp.float32),
                pltpu.VMEM((CH, HF), jnp.float32),
                pltpu.VMEM((CH, HF), jnp.float32),
                pltpu.VMEM((ZR, HF), jnp.float32),
                pltpu.VMEM_SHARED((NP, HF), jnp.float32),
                pltpu.VMEM_SHARED((NP, HF), jnp.float32),
            ] + [pltpu.SemaphoreType.DMA] * 9,
        )(_spmm_body)
    zeros = jnp.zeros((ZR, HF), jnp.float32)
    return _SC_BUILT["spmm"](hlo, hhi, row3, col3, zeros)


def _prep_body(x_ref, da_ref, W1_ref, b1_ref, W2_ref, b2_ref, Wc0_ref,
               h_ref, hlo_ref, hhi_ref, dinv_ref):
    xv = x_ref[0, 0, :]
    t = jnp.maximum(xv[:, None] * W1_ref[0][None, :] + b1_ref[0][None, :], 0.0)
    h = jnp.dot(t, W2_ref[...], preferred_element_type=jnp.float32)
    h = h + b2_ref[0][None, :]
    deg = da_ref[0, :, 0] + da_ref[1, :, 0] + 1.0
    dinv = lax.rsqrt(deg)
    hw = jnp.dot(h, Wc0_ref[...], preferred_element_type=jnp.float32)
    h_ref[0] = h
    hws = dinv[:, None] * hw
    hlo_ref[0] = hws[:, :HF]
    hhi_ref[0] = hws[:, HF:]
    dinv_ref[...] = dinv


def _mid_body(alo_ref, ahi_ref, hlo_ref, hhi_ref, h_ref, dinv_ref, g_ref,
              bt_ref, bc_ref, Wc1_ref, h1_ref, h1lo_ref, h1hi_ref):
    dinv = dinv_ref[...]
    s = jnp.concatenate([alo_ref[0] + hlo_ref[0], ahi_ref[0] + hhi_ref[0]],
                        axis=-1)
    t = dinv[:, None] * s + bc_ref[0][None, :]
    t = t * (g_ref[0][None, :] * _BN_SCALE) + bt_ref[0][None, :]
    h1 = jnp.maximum(t, 0.0) + h_ref[0]
    hw1 = jnp.dot(h1, Wc1_ref[...], preferred_element_type=jnp.float32)
    h1_ref[0] = h1
    hws1 = dinv[:, None] * hw1
    h1lo_ref[0] = hws1[:, :HF]
    h1hi_ref[0] = hws1[:, HF:]


def _pool_body(alo_ref, ahi_ref, hlo_ref, hhi_ref, h1_ref, dinv_ref, g_ref,
               bt_ref, bc_ref, out_ref):
    n = pl.program_id(1)
    dinv = dinv_ref[...]
    s = jnp.concatenate([alo_ref[0] + hlo_ref[0], ahi_ref[0] + hhi_ref[0]],
                        axis=-1)
    t = dinv[:, None] * s + bc_ref[0][None, :]
    t = t * (g_ref[0][None, :] * _BN_SCALE) + bt_ref[0][None, :]
    h2 = jnp.maximum(t, 0.0) + h1_ref[0]
    iot = lax.broadcasted_iota(jnp.int32, (BN, 1), 0)
    h2 = jnp.where(iot < (N_ - n * BN), h2, 0.0)
    part = jnp.sum(h2, axis=0)

    @pl.when(n == 0)
    def _():
        out_ref[0, 0, :] = part

    @pl.when(n > 0)
    def _():
        out_ref[0, 0, :] = out_ref[0, 0, :] + part


def _proj_body(p_ref, Wp_ref, bp_ref, z_ref):
    z = jnp.dot(p_ref[...] * (1.0 / N_), Wp_ref[...],
                preferred_element_type=jnp.float32)
    z_ref[...] = z + bp_ref[0][None, :]


def _full(shape):
    return pl.BlockSpec(shape, lambda b, n: tuple(0 for _ in shape))


def kernel(x, edge_index, W1, b1, W2, b2, Wc0, bc0, Wc1, bc1, g0, bt0,
                 g1, bt1, Wp, bp):
    f32 = jnp.float32
    xp = jnp.pad(x, ((0, 0), (0, NP - N_))).reshape(B_, 1, NP)
    row = jnp.pad(edge_index[0], (0, EP - E_), constant_values=NP - 1)
    col = jnp.pad(edge_index[1], (0, EP - E_), constant_values=NP - 1)
    row3 = row.reshape(NS, NCH, CH)
    col3 = col.reshape(NS, NCH, CH)
    b1r, b2r = b1.reshape(1, -1), b2.reshape(1, -1)
    bc0r, bc1r = bc0.reshape(1, -1), bc1.reshape(1, -1)
    g0r, g1r = g0.reshape(1, -1), g1.reshape(1, -1)
    bt0r, bt1r = bt0.reshape(1, -1), bt1.reshape(1, -1)
    bpr = bp.reshape(1, -1)

    degacc = _deg_sc(col).reshape(NC, NP, 16)

    grid = (B_, NP // BN)
    node3 = pl.BlockSpec((1, BN, H_), lambda b, n: (b, n, 0))
    half3 = pl.BlockSpec((1, BN, HF), lambda b, n: (b, n, 0))
    dinv_spec = pl.BlockSpec((BN,), lambda b, n: (n,))
    half_sds = jax.ShapeDtypeStruct((B_, NP, HF), f32)
    h, hws0lo, hws0hi, dinv = pl.pallas_call(
        _prep_body,
        grid=grid,
        in_specs=[
            pl.BlockSpec((1, 1, BN), lambda b, n: (b, 0, n)),
            pl.BlockSpec((NC, BN, 16), lambda b, n: (0, n, 0)),
            _full((1, 64)), _full((1, 64)), _full((64, H_)), _full((1, H_)),
            _full((H_, H_)),
        ],
        out_specs=[node3, half3, half3, dinv_spec],
        out_shape=[
            jax.ShapeDtypeStruct((B_, NP, H_), f32),
            half_sds, half_sds,
            jax.ShapeDtypeStruct((NP,), f32),
        ],
    )(xp, degacc, W1, b1r, W2, b2r, Wc0)

    a0lo, a0hi = _spmm_sc(hws0lo.reshape(B_ * NP, HF),
                          hws0hi.reshape(B_ * NP, HF), row3, col3)
    a0lo = a0lo.reshape(B_, NP, HF)
    a0hi = a0hi.reshape(B_, NP, HF)

    h1, hws1lo, hws1hi = pl.pallas_call(
        _mid_body,
        grid=grid,
        in_specs=[
            half3, half3, half3, half3, node3, dinv_spec,
            _full((1, H_)), _full((1, H_)), _full((1, H_)), _full((H_, H_)),
        ],
        out_specs=[node3, half3, half3],
        out_shape=[
            jax.ShapeDtypeStruct((B_, NP, H_), f32),
            half_sds, half_sds,
        ],
    )(a0lo, a0hi, hws0lo, hws0hi, h, dinv, g0r, bt0r, bc0r, Wc1)

    a1lo, a1hi = _spmm_sc(hws1lo.reshape(B_ * NP, HF),
                          hws1hi.reshape(B_ * NP, HF), row3, col3)
    a1lo = a1lo.reshape(B_, NP, HF)
    a1hi = a1hi.reshape(B_, NP, HF)

    pooled = pl.pallas_call(
        _pool_body,
        grid=grid,
        in_specs=[
            half3, half3, half3, half3, node3, dinv_spec,
            _full((1, H_)), _full((1, H_)), _full((1, H_)),
        ],
        out_specs=pl.BlockSpec((1, 1, H_), lambda b, n: (b, 0, 0)),
        out_shape=jax.ShapeDtypeStruct((B_, 1, H_), f32),
    )(a1lo, a1hi, hws1lo, hws1hi, h1, dinv, g1r, bt1r, bc1r)

    z = pl.pallas_call(
        _proj_body,
        grid=(1, 1),
        in_specs=[_full((B_, H_)), _full((H_, H_)), _full((1, H_))],
        out_specs=_full((B_, H_)),
        out_shape=jax.ShapeDtypeStruct((B_, H_), f32),
    )(pooled.reshape(B_, H_), Wp, bpr)
    return z
